# baseline (speedup 1.0000x reference)
import functools

import jax
import jax.numpy as jnp
from jax import lax
from jax.experimental import pallas as pl
from jax.experimental.pallas import tpu as pltpu
from jax.experimental.pallas import tpu_sc as plsc

N_NODES = 50000
N_DIM = 64
N_EDGES = 800000
N_LAYERS = 3

NC = 2
NS = 16
HALF = N_DIM // NC
NPAD = 50048
ROWS_PER_TILE = NPAD // NS
OUT_CHUNK = 136
N_OUT_CHUNKS = ROWS_PER_TILE // OUT_CHUNK

G = 128
NSLOTS = 4
SUPER = 2048
GPS = SUPER // G
SUPERS_PER_TILE = 25
EDGES_PAD = NS * SUPERS_PER_TILE * SUPER
GROUPS_TOTAL = EDGES_PAD // G
GROUPS_PER_TILE = GROUPS_TOTAL // NS

_f32 = jnp.float32
_i32 = jnp.int32


def _gacn_body(emb2, srcg, dstg, wflat, out, s1, s2, s3,
               srcsb, dstsb, wsb, rows, bounce, acc, gsem, ssem, lsem, zsem):
    c = lax.axis_index("c").astype(_i32)
    s = lax.axis_index("s").astype(_i32)
    node_base = s * _i32(ROWS_PER_TILE)
    cbase = c * _i32(NPAD)

    def rslot(slot):
        return rows.at[pl.ds(slot * G, G)]

    @pl.loop(0, OUT_CHUNK)
    def _z(r):
        r = r.astype(_i32)
        bounce[r, pl.ds(0, 16)] = jnp.zeros((16,), _f32)
        bounce[r, pl.ds(16, 16)] = jnp.zeros((16,), _f32)

    def zero_acc_slice():
        for k in range(N_OUT_CHUNKS):
            pltpu.sync_copy(
                bounce, acc.at[pl.ds(node_base + k * OUT_CHUNK, OUT_CHUNK)])

    zero_acc_slice()
    plsc.subcore_barrier()

    def fire_gather(src_tab, k, slot):
        pltpu.async_copy(src_tab.at[srcsb.at[k]], rslot(slot), gsem.at[slot])

    def wait_gather(slot):
        pltpu.make_async_copy(
            emb2.at[pl.ds(0, G)], rslot(slot), gsem.at[slot]).wait()

    def fire_scatter(k, slot):
        pltpu.async_copy(rslot(slot), acc.at[dstsb.at[k]], ssem.at[slot],
                         add=True)

    def wait_scatter(slot):
        pltpu.make_async_copy(
            emb2.at[pl.ds(0, G)], rslot(slot), ssem.at[slot]).wait()

    def multiply(k, slot):
        base = slot * G

        @pl.loop(0, G // 16)
        def _mul(j):
            j = j.astype(_i32)
            wrow = wsb[pl.ds(k * G + j * 16, 16)]
            for i in range(16):
                e = base + j * 16 + i
                wv = jnp.full((16,), wrow[i], _f32)
                rows[e, pl.ds(0, 16)] = rows[e, pl.ds(0, 16)] * wv
                rows[e, pl.ds(16, 16)] = rows[e, pl.ds(16, 16)] * wv

    stages = [s1, s2, s3]
    sources = [emb2] + stages
    for l in range(N_LAYERS):
        src_tab = sources[l]

        @pl.loop(0, SUPERS_PER_TILE)
        def _super(u):
            u = u.astype(_i32)
            gb = s * _i32(GROUPS_PER_TILE) + u * _i32(GPS)
            pltpu.sync_copy(
                srcg.at[pl.ds(c * _i32(GROUPS_TOTAL) + gb, GPS)], srcsb)
            pltpu.sync_copy(dstg.at[pl.ds(gb, GPS)], dstsb)
            pltpu.sync_copy(wflat.at[pl.ds(gb * _i32(G), SUPER)], wsb)
            for k in range(NSLOTS - 1):
                fire_gather(src_tab, k, k)
            for k in range(GPS):
                slot = k % NSLOTS
                if k + NSLOTS - 1 < GPS:
                    p = (k + NSLOTS - 1) % NSLOTS
                    if k >= 1:
                        wait_scatter(p)
                    fire_gather(src_tab, k + NSLOTS - 1, p)
                wait_gather(slot)
                multiply(k, slot)
                fire_scatter(k, slot)
            for slot in range(NSLOTS):
                wait_scatter(slot)

        plsc.subcore_barrier()
        stage = stages[l]
        pp = rows.at[pl.ds(0, OUT_CHUNK)]
        for k in range(N_OUT_CHUNKS):
            sl = pl.ds(node_base + k * OUT_CHUNK, OUT_CHUNK)
            gsl = pl.ds(cbase + node_base + k * OUT_CHUNK, OUT_CHUNK)
            pltpu.sync_copy(acc.at[sl], pp)
            pltpu.sync_copy(pp, stage.at[gsl])
        if l + 1 < N_LAYERS:
            zero_acc_slice()
        plsc.subcore_barrier()

    qv = jnp.full((16,), 0.25, dtype=_f32)
    b = [rows.at[pl.ds(j * OUT_CHUNK, OUT_CHUNK)] for j in range(3)]
    for k in range(N_OUT_CHUNKS):
        goff = cbase + node_base + k * OUT_CHUNK
        gsl = pl.ds(goff, OUT_CHUNK)
        pltpu.sync_copy(emb2.at[gsl], bounce)
        for j, st in enumerate(stages):
            pltpu.sync_copy(st.at[gsl], b[j])

        @pl.loop(0, OUT_CHUNK)
        def _mean(r):
            r = r.astype(_i32)
            for h in (0, 16):
                v = (bounce[r, pl.ds(h, 16)] + b[0][r, pl.ds(h, 16)]
                     + b[1][r, pl.ds(h, 16)] + b[2][r, pl.ds(h, 16)])
                bounce[r, pl.ds(h, 16)] = v * qv

        pltpu.sync_copy(bounce, out.at[gsl])


_HT = jax.ShapeDtypeStruct((NC * NPAD, HALF), _f32)

_gacn = functools.partial(
    pl.kernel,
    out_type=(_HT, _HT, _HT, _HT),
    mesh=plsc.VectorSubcoreMesh(core_axis_name="c", subcore_axis_name="s"),
    compiler_params=pltpu.CompilerParams(needs_layout_passes=False,
                                         use_tc_tiling_on_sc=False),
    scratch_types=(
        pltpu.VMEM((GPS, G), _i32),
        pltpu.VMEM((GPS, G), _i32),
        pltpu.VMEM((SUPER,), _f32),
        pltpu.VMEM((NSLOTS * G, HALF), _f32),
        pltpu.VMEM((OUT_CHUNK, HALF), _f32),
        pltpu.VMEM_SHARED((NPAD, HALF), _f32),
        pltpu.SemaphoreType.DMA((NSLOTS,)),
        pltpu.SemaphoreType.DMA((NSLOTS,)),
        pltpu.SemaphoreType.DMA,
        pltpu.SemaphoreType.DMA,
    ),
)(_gacn_body)


def kernel(emb, edge_index, edge_weight):
    emb = emb.astype(_f32)
    dst = edge_index[0].astype(_i32)
    src = edge_index[1].astype(_i32)
    w = edge_weight.astype(_f32)
    prev_x64 = bool(jax.config.jax_enable_x64)
    jax.config.update("jax_enable_x64", False)
    try:
        return _run(emb, dst, src, w)
    finally:
        jax.config.update("jax_enable_x64", prev_x64)


def _run(emb, dst, src, w):
    pad = EDGES_PAD - N_EDGES
    src = jnp.concatenate([src, jnp.zeros((pad,), _i32)])
    dst = jnp.concatenate([dst, jnp.arange(pad, dtype=_i32) % N_NODES])
    w = jnp.concatenate([w, jnp.zeros((pad,), _f32)])
    srcg = jnp.concatenate([src, src + NPAD]).reshape(2 * GROUPS_TOTAL, G)
    dstg = dst.reshape(GROUPS_TOTAL, G)
    emb2 = (emb.reshape(N_NODES, NC, HALF).transpose(1, 0, 2)
            .reshape(NC, N_NODES, HALF))
    emb2 = jnp.concatenate(
        [emb2, jnp.zeros((NC, NPAD - N_NODES, HALF), _f32)], axis=1)
    emb2 = emb2.reshape(NC * NPAD, HALF)
    out, _e1, _e2, _e3 = _gacn(emb2, srcg, dstg, w)
    return (out.reshape(NC, NPAD, HALF)[:, :N_NODES]
            .transpose(1, 0, 2).reshape(N_NODES, N_DIM))


# speedup vs baseline: 1.0725x; 1.0725x over previous
import functools

import jax
import jax.numpy as jnp
from jax import lax
from jax.experimental import pallas as pl
from jax.experimental.pallas import tpu as pltpu
from jax.experimental.pallas import tpu_sc as plsc

N_NODES = 50000
N_DIM = 64
N_EDGES = 800000
N_LAYERS = 3

NC = 2
NS = 16
HALF = N_DIM // NC
NPAD = 50048
ROWS_PER_TILE = NPAD // NS
OUT_CHUNK = 136
N_OUT_CHUNKS = ROWS_PER_TILE // OUT_CHUNK

G = 128
NSLOTS = 4
SUPER = 2048
GPS = SUPER // G
SUPERS_PER_TILE = 25
EDGES_PAD = NS * SUPERS_PER_TILE * SUPER
GROUPS_TOTAL = EDGES_PAD // G
GROUPS_PER_TILE = GROUPS_TOTAL // NS

_f32 = jnp.float32
_i32 = jnp.int32


def _gacn_body(emb2, srcg, dstg, wflat, out, s1, s2, s3,
               srcsb, dstsb, wsb, rows, bounce, acc, gsem, ssem, lsem, zsem):
    c = lax.axis_index("c").astype(_i32)
    s = lax.axis_index("s").astype(_i32)
    node_base = s * _i32(ROWS_PER_TILE)
    cbase = c * _i32(NPAD)

    def rslot(slot):
        return rows.at[pl.ds(slot * G, G)]

    @pl.loop(0, OUT_CHUNK)
    def _z(r):
        r = r.astype(_i32)
        bounce[r, pl.ds(0, 16)] = jnp.zeros((16,), _f32)
        bounce[r, pl.ds(16, 16)] = jnp.zeros((16,), _f32)

    def zero_acc_slice():
        for k in range(N_OUT_CHUNKS):
            pltpu.sync_copy(
                bounce, acc.at[pl.ds(node_base + k * OUT_CHUNK, OUT_CHUNK)])

    zero_acc_slice()
    plsc.subcore_barrier()

    def fire_gather(src_tab, k, slot):
        pltpu.async_copy(src_tab.at[srcsb.at[k]], rslot(slot), gsem.at[slot])

    def wait_gather(slot):
        pltpu.make_async_copy(
            emb2.at[pl.ds(0, G)], rslot(slot), gsem.at[slot]).wait()

    def fire_scatter(k, slot):
        pltpu.async_copy(rslot(slot), acc.at[dstsb.at[k]], ssem.at[slot],
                         add=True)

    def wait_scatter(slot):
        pltpu.make_async_copy(
            emb2.at[pl.ds(0, G)], rslot(slot), ssem.at[slot]).wait()

    def multiply(k, slot):
        base = slot * G

        @pl.loop(0, G // 16)
        def _mul(j):
            j = j.astype(_i32)
            wrow = wsb[pl.ds(k * G + j * 16, 16)]
            for i in range(16):
                e = base + j * 16 + i
                wv = jnp.full((16,), wrow[i], _f32)
                rows[e, pl.ds(0, 16)] = rows[e, pl.ds(0, 16)] * wv
                rows[e, pl.ds(16, 16)] = rows[e, pl.ds(16, 16)] * wv

    stages = [s1, s2, s3]
    sources = [emb2] + stages
    for l in range(N_LAYERS):
        src_tab = sources[l]

        @pl.loop(0, SUPERS_PER_TILE)
        def _super(u):
            u = u.astype(_i32)
            gb = s * _i32(GROUPS_PER_TILE) + u * _i32(GPS)
            pltpu.async_copy(
                srcg.at[pl.ds(c * _i32(GROUPS_TOTAL) + gb, GPS)], srcsb,
                lsem.at[0])
            pltpu.async_copy(dstg.at[pl.ds(gb, GPS)], dstsb, lsem.at[1])
            pltpu.async_copy(wflat.at[pl.ds(gb * _i32(G), SUPER)], wsb,
                             lsem.at[2])
            pltpu.make_async_copy(srcg.at[pl.ds(0, GPS)], srcsb,
                                  lsem.at[0]).wait()
            pltpu.make_async_copy(dstg.at[pl.ds(0, GPS)], dstsb,
                                  lsem.at[1]).wait()
            pltpu.make_async_copy(wflat.at[pl.ds(0, SUPER)], wsb,
                                  lsem.at[2]).wait()
            for k in range(NSLOTS - 1):
                fire_gather(src_tab, k, k)
            for k in range(GPS):
                slot = k % NSLOTS
                if k + NSLOTS - 1 < GPS:
                    p = (k + NSLOTS - 1) % NSLOTS
                    if k >= 1:
                        wait_scatter(p)
                    fire_gather(src_tab, k + NSLOTS - 1, p)
                wait_gather(slot)
                multiply(k, slot)
                fire_scatter(k, slot)
            for slot in range(NSLOTS):
                wait_scatter(slot)

        plsc.subcore_barrier()
        stage = stages[l]
        pp = rows.at[pl.ds(0, OUT_CHUNK)]
        for k in range(N_OUT_CHUNKS):
            sl = pl.ds(node_base + k * OUT_CHUNK, OUT_CHUNK)
            gsl = pl.ds(cbase + node_base + k * OUT_CHUNK, OUT_CHUNK)
            pltpu.sync_copy(acc.at[sl], pp)
            pltpu.sync_copy(pp, stage.at[gsl])
        if l + 1 < N_LAYERS:
            zero_acc_slice()
        plsc.subcore_barrier()

    qv = jnp.full((16,), 0.25, dtype=_f32)
    b = [rows.at[pl.ds(j * OUT_CHUNK, OUT_CHUNK)] for j in range(3)]
    for k in range(N_OUT_CHUNKS):
        goff = cbase + node_base + k * OUT_CHUNK
        gsl = pl.ds(goff, OUT_CHUNK)
        pltpu.async_copy(emb2.at[gsl], bounce, lsem.at[3])
        for j, st in enumerate(stages):
            pltpu.async_copy(st.at[gsl], b[j], lsem.at[j])
        pltpu.make_async_copy(emb2.at[pl.ds(0, OUT_CHUNK)], bounce,
                              lsem.at[3]).wait()
        for j in range(3):
            pltpu.make_async_copy(emb2.at[pl.ds(0, OUT_CHUNK)], b[j],
                                  lsem.at[j]).wait()

        @pl.loop(0, OUT_CHUNK)
        def _mean(r):
            r = r.astype(_i32)
            for h in (0, 16):
                v = (bounce[r, pl.ds(h, 16)] + b[0][r, pl.ds(h, 16)]
                     + b[1][r, pl.ds(h, 16)] + b[2][r, pl.ds(h, 16)])
                bounce[r, pl.ds(h, 16)] = v * qv

        pltpu.sync_copy(bounce, out.at[gsl])


_HT = jax.ShapeDtypeStruct((NC * NPAD, HALF), _f32)

_gacn = functools.partial(
    pl.kernel,
    out_type=(_HT, _HT, _HT, _HT),
    mesh=plsc.VectorSubcoreMesh(core_axis_name="c", subcore_axis_name="s"),
    compiler_params=pltpu.CompilerParams(needs_layout_passes=False,
                                         use_tc_tiling_on_sc=False),
    scratch_types=(
        pltpu.VMEM((GPS, G), _i32),
        pltpu.VMEM((GPS, G), _i32),
        pltpu.VMEM((SUPER,), _f32),
        pltpu.VMEM((NSLOTS * G, HALF), _f32),
        pltpu.VMEM((OUT_CHUNK, HALF), _f32),
        pltpu.VMEM_SHARED((NPAD, HALF), _f32),
        pltpu.SemaphoreType.DMA((NSLOTS,)),
        pltpu.SemaphoreType.DMA((NSLOTS,)),
        pltpu.SemaphoreType.DMA((NSLOTS,)),
        pltpu.SemaphoreType.DMA((NSLOTS,)),
    ),
)(_gacn_body)


def kernel(emb, edge_index, edge_weight):
    emb = emb.astype(_f32)
    dst = edge_index[0].astype(_i32)
    src = edge_index[1].astype(_i32)
    w = edge_weight.astype(_f32)
    prev_x64 = bool(jax.config.jax_enable_x64)
    jax.config.update("jax_enable_x64", False)
    try:
        return _run(emb, dst, src, w)
    finally:
        jax.config.update("jax_enable_x64", prev_x64)


def _run(emb, dst, src, w):
    pad = EDGES_PAD - N_EDGES
    src = jnp.concatenate([src, jnp.zeros((pad,), _i32)])
    dst = jnp.concatenate([dst, jnp.arange(pad, dtype=_i32) % N_NODES])
    w = jnp.concatenate([w, jnp.zeros((pad,), _f32)])
    srcg = jnp.concatenate([src, src + NPAD]).reshape(2 * GROUPS_TOTAL, G)
    dstg = dst.reshape(GROUPS_TOTAL, G)
    emb2 = (emb.reshape(N_NODES, NC, HALF).transpose(1, 0, 2)
            .reshape(NC, N_NODES, HALF))
    emb2 = jnp.concatenate(
        [emb2, jnp.zeros((NC, NPAD - N_NODES, HALF), _f32)], axis=1)
    emb2 = emb2.reshape(NC * NPAD, HALF)
    out, _e1, _e2, _e3 = _gacn(emb2, srcg, dstg, w)
    return (out.reshape(NC, NPAD, HALF)[:, :N_NODES]
            .transpose(1, 0, 2).reshape(N_NODES, N_DIM))
